# baseline (speedup 1.0000x reference)
import jax
import jax.numpy as jnp
from jax import lax
from jax.experimental import pallas as pl
from jax.experimental.pallas import tpu as pltpu

B, N_FEAT, D = 16384, 100, 128
NP1 = N_FEAT + 1
BB = 256
KQ = 8
NSTEPS = B // BB


def _tok_body(xn_ref, w_ref, b_ref, o_hbm, o_buf, sems):
    i = pl.program_id(0)
    slot = lax.rem(i, 2)

    @pl.when(i >= 2)
    def _wait_prev():
        for k in range(KQ):
            pltpu.make_async_copy(
                o_buf.at[slot, pl.ds(k * (BB // KQ), BB // KQ)],
                o_hbm.at[pl.ds((i - 2) * BB + k * (BB // KQ), BB // KQ)],
                sems.at[slot, k],
            ).wait()

    xn = xn_ref[...]
    o_buf[slot] = xn[:, :, None] * w_ref[...][None] + b_ref[...][None]

    def issue(j, carry):
        base = j * KQ
        for k in range(KQ):
            r = base + k
            pltpu.make_async_copy(
                o_buf.at[slot, r],
                o_hbm.at[i * BB + r],
                sems.at[slot, k],
            ).start()
        return carry

    lax.fori_loop(0, BB // KQ, issue, 0)

    @pl.when(i == NSTEPS - 1)
    def _drain():
        for s in range(2):
            base_i = NSTEPS - 2 if s == (NSTEPS - 2) % 2 else NSTEPS - 1
            for k in range(KQ):
                pltpu.make_async_copy(
                    o_buf.at[s, pl.ds(k * (BB // KQ), BB // KQ)],
                    o_hbm.at[pl.ds(base_i * BB + k * (BB // KQ), BB // KQ)],
                    sems.at[s, k],
                ).wait()


def kernel(x, numerical_weight, numerical_bias):
    ones = jnp.ones((x.shape[0], 1), dtype=x.dtype)
    xn = jnp.concatenate([ones, x], axis=1)
    zero = jnp.zeros((1, D), dtype=numerical_bias.dtype)
    bias_p = jnp.concatenate([zero, numerical_bias], axis=0)

    return pl.pallas_call(
        _tok_body,
        grid=(NSTEPS,),
        in_specs=[
            pl.BlockSpec((BB, NP1), lambda i: (i, 0)),
            pl.BlockSpec((NP1, D), lambda i: (0, 0)),
            pl.BlockSpec((NP1, D), lambda i: (0, 0)),
        ],
        out_specs=pl.BlockSpec(memory_space=pltpu.MemorySpace.HBM),
        out_shape=jax.ShapeDtypeStruct((B, NP1, D), x.dtype),
        scratch_shapes=[
            pltpu.VMEM((2, BB, NP1, D), jnp.float32),
            pltpu.SemaphoreType.DMA((2, KQ)),
        ],
        compiler_params=pltpu.CompilerParams(
            dimension_semantics=("arbitrary",),
        ),
    )(xn, numerical_weight, bias_p)
